# baseline (speedup 1.0000x reference)
import functools

import jax
import jax.numpy as jnp
from jax import lax
from jax.experimental import pallas as pl
from jax.experimental.pallas import tpu as pltpu
from jax.experimental.pallas import tpu_sc as plsc

B, L = 1024, 200
N = B * L
D = 128
N_TAGS = 45

NCHUNK = 5
CL = L // NCHUNK
CN = CL * B

_info = plsc.get_sparse_core_info()
NC, NS = _info.num_cores, _info.num_subcores
NW = NC * NS
PER_W = CN // NW
CH = 128
N_CH = PER_W // CH


NBUF = 4


def _sc_gather_kernel(idx_hbm, table_hbm, out_hbm, idx_v, *scr):
    bufs, gsems, osems = scr[:NBUF], scr[NBUF:2 * NBUF], scr[2 * NBUF:]
    wid = lax.axis_index("s") * NC + lax.axis_index("c")
    base = wid * PER_W
    pltpu.sync_copy(idx_hbm.at[pl.ds(base, PER_W)], idx_v)

    def start_gather(j):
        b = j % NBUF
        pltpu.async_copy(
            table_hbm.at[idx_v.at[pl.ds(j * CH, CH)]], bufs[b], gsems[b]
        )

    for j in range(min(NBUF, N_CH)):
        start_gather(j)

    for j in range(N_CH):
        b = j % NBUF
        pltpu.make_async_copy(
            table_hbm.at[idx_v.at[pl.ds(j * CH, CH)]], bufs[b], gsems[b]
        ).wait()
        pltpu.async_copy(bufs[b], out_hbm.at[pl.ds(base + j * CH, CH)], osems[b])
        nj = j + 1
        if NBUF <= nj < N_CH:
            bn = nj % NBUF
            pltpu.make_async_copy(
                bufs[bn],
                out_hbm.at[pl.ds(base + (nj - NBUF) * CH, CH)],
                osems[bn],
            ).wait()
            start_gather(nj)

    for j in range(max(0, N_CH - NBUF), N_CH):
        b = j % NBUF
        pltpu.make_async_copy(
            bufs[b], out_hbm.at[pl.ds(base + j * CH, CH)], osems[b]
        ).wait()


def _sc_gather(idx, table):
    mesh = plsc.VectorSubcoreMesh(core_axis_name="c", subcore_axis_name="s")
    k = functools.partial(
        pl.kernel,
        mesh=mesh,
        out_type=jax.ShapeDtypeStruct((CN, D), jnp.float32),
        scratch_types=[
            pltpu.VMEM((PER_W,), jnp.int32),
        ]
        + [pltpu.VMEM((CH, D), jnp.float32) for _ in range(NBUF)]
        + [pltpu.SemaphoreType.DMA for _ in range(2 * NBUF)],
    )(_sc_gather_kernel)
    return k(idx, table)


BL = 8


def _mlp_body(g_ref, w1t_ref, b1_ref, w2_ref, b2_ref, out_ref):
    h = jnp.maximum(
        jnp.dot(
            g_ref[...], w1t_ref[...], preferred_element_type=jnp.float32
        )
        + b1_ref[...],
        0.0,
    )
    for j in range(BL):
        ot = (
            lax.dot_general(
                w2_ref[...], h[j * B:(j + 1) * B, :], (((1,), (1,)), ((), ())),
                preferred_element_type=jnp.float32,
            )
            + b2_ref[...]
        )
        out_ref[:, j, :] = ot


def _mlp_kernel_first(g_ref, w1t_ref, b1_ref, w2_ref, b2_ref, out_ref):
    _mlp_body(g_ref, w1t_ref, b1_ref, w2_ref, b2_ref, out_ref)


def _mlp_kernel_acc(g_ref, w1t_ref, b1_ref, w2_ref, b2_ref, acc_ref, out_ref):
    _mlp_body(g_ref, w1t_ref, b1_ref, w2_ref, b2_ref, out_ref)


def _tc_mlp_chunk(k, g, w1t, b1, w2, b2c, acc):
    k0 = k * (CL // BL)
    in_specs = [
        pl.BlockSpec((BL * B, D), lambda i: (i, 0)),
        pl.BlockSpec((D, D), lambda i: (0, 0)),
        pl.BlockSpec((1, D), lambda i: (0, 0)),
        pl.BlockSpec((N_TAGS, D), lambda i: (0, 0)),
        pl.BlockSpec((N_TAGS, 1), lambda i: (0, 0)),
    ]
    args = (g, w1t, b1, w2, b2c)
    body = _mlp_kernel_first
    kwargs = {}
    if acc is not None:
        in_specs.append(pl.BlockSpec(memory_space=pl.ANY))
        args = args + (acc,)
        body = _mlp_kernel_acc
        kwargs = dict(input_output_aliases={5: 0})
    return pl.pallas_call(
        body,
        grid=(CL // BL,),
        in_specs=in_specs,
        out_specs=pl.BlockSpec((N_TAGS, BL, B), lambda i: (0, k0 + i, 0)),
        out_shape=jax.ShapeDtypeStruct((N_TAGS, L, B), jnp.float32),
        **kwargs,
    )(*args)


def kernel(x, emb, W1, b1, W2, b2):
    idx = x.T.reshape(-1).astype(jnp.int32)
    w1t = W1.T
    b1r = b1.reshape(1, D)
    b2c = b2.reshape(N_TAGS, 1)
    gs = [_sc_gather(lax.dynamic_slice(idx, (k * CN,), (CN,)), emb)
          for k in range(NCHUNK)]
    out_t = None
    for k in range(NCHUNK):
        out_t = _tc_mlp_chunk(k, gs[k], w1t, b1r, W2, b2c, out_t)
    return lax.transpose(out_t, (2, 1, 0))


# speedup vs baseline: 1.0600x; 1.0600x over previous
import functools

import jax
import jax.numpy as jnp
from jax import lax
from jax.experimental import pallas as pl
from jax.experimental.pallas import tpu as pltpu
from jax.experimental.pallas import tpu_sc as plsc

B, L = 1024, 200
N = B * L
D = 128
N_TAGS = 45

NCHUNK = 5
CL = L // NCHUNK
CN = CL * B

_info = plsc.get_sparse_core_info()
NC, NS = _info.num_cores, _info.num_subcores
NW = NC * NS
PER_W = CN // NW
CH = 128
N_CH = PER_W // CH


NBUF = 4
DP = 2


def _sc_gather_kernel(idx_hbm, table_hbm, out_hbm, idx_v, *scr):
    bufs, gsems, osems = scr[:NBUF], scr[NBUF:2 * NBUF], scr[2 * NBUF:]
    wid = lax.axis_index("s") * NC + lax.axis_index("c")
    base = wid * PER_W
    pltpu.sync_copy(idx_hbm.at[pl.ds(base, PER_W)], idx_v)

    def start_gather(j):
        b = j % NBUF
        pltpu.async_copy(
            table_hbm.at[idx_v.at[pl.ds(j * CH, CH)]], bufs[b], gsems[b]
        )

    for j in range(min(DP, N_CH)):
        start_gather(j)

    for j in range(N_CH):
        b = j % NBUF
        pltpu.make_async_copy(
            table_hbm.at[idx_v.at[pl.ds(j * CH, CH)]], bufs[b], gsems[b]
        ).wait()
        pltpu.async_copy(bufs[b], out_hbm.at[pl.ds(base + j * CH, CH)], osems[b])
        nj = j + DP
        if DP <= nj < N_CH:
            bn = nj % NBUF
            if nj >= NBUF:
                pltpu.make_async_copy(
                    bufs[bn],
                    out_hbm.at[pl.ds(base + (nj - NBUF) * CH, CH)],
                    osems[bn],
                ).wait()
            start_gather(nj)

    for j in range(max(0, N_CH - NBUF), N_CH):
        b = j % NBUF
        pltpu.make_async_copy(
            bufs[b], out_hbm.at[pl.ds(base + j * CH, CH)], osems[b]
        ).wait()


def _sc_gather(idx, table):
    mesh = plsc.VectorSubcoreMesh(core_axis_name="c", subcore_axis_name="s")
    k = functools.partial(
        pl.kernel,
        mesh=mesh,
        out_type=jax.ShapeDtypeStruct((CN, D), jnp.float32),
        scratch_types=[
            pltpu.VMEM((PER_W,), jnp.int32),
        ]
        + [pltpu.VMEM((CH, D), jnp.float32) for _ in range(NBUF)]
        + [pltpu.SemaphoreType.DMA for _ in range(2 * NBUF)],
    )(_sc_gather_kernel)
    return k(idx, table)


BL = 8


def _mlp_body(g_ref, w1t_ref, b1_ref, w2_ref, b2_ref, out_ref):
    h = jnp.maximum(
        jnp.dot(
            g_ref[...], w1t_ref[...], preferred_element_type=jnp.float32
        )
        + b1_ref[...],
        0.0,
    )
    for j in range(BL):
        ot = (
            lax.dot_general(
                w2_ref[...], h[j * B:(j + 1) * B, :], (((1,), (1,)), ((), ())),
                preferred_element_type=jnp.float32,
            )
            + b2_ref[...]
        )
        out_ref[:, j, :] = ot


def _mlp_kernel_first(g_ref, w1t_ref, b1_ref, w2_ref, b2_ref, out_ref):
    _mlp_body(g_ref, w1t_ref, b1_ref, w2_ref, b2_ref, out_ref)


def _mlp_kernel_acc(g_ref, w1t_ref, b1_ref, w2_ref, b2_ref, acc_ref, out_ref):
    _mlp_body(g_ref, w1t_ref, b1_ref, w2_ref, b2_ref, out_ref)


def _tc_mlp_chunk(k, g, w1t, b1, w2, b2c, acc):
    k0 = k * (CL // BL)
    in_specs = [
        pl.BlockSpec((BL * B, D), lambda i: (i, 0)),
        pl.BlockSpec((D, D), lambda i: (0, 0)),
        pl.BlockSpec((1, D), lambda i: (0, 0)),
        pl.BlockSpec((N_TAGS, D), lambda i: (0, 0)),
        pl.BlockSpec((N_TAGS, 1), lambda i: (0, 0)),
    ]
    args = (g, w1t, b1, w2, b2c)
    body = _mlp_kernel_first
    kwargs = {}
    if acc is not None:
        in_specs.append(pl.BlockSpec(memory_space=pl.ANY))
        args = args + (acc,)
        body = _mlp_kernel_acc
        kwargs = dict(input_output_aliases={5: 0})
    return pl.pallas_call(
        body,
        grid=(CL // BL,),
        in_specs=in_specs,
        out_specs=pl.BlockSpec((N_TAGS, BL, B), lambda i: (0, k0 + i, 0)),
        out_shape=jax.ShapeDtypeStruct((N_TAGS, L, B), jnp.float32),
        **kwargs,
    )(*args)


def kernel(x, emb, W1, b1, W2, b2):
    idx = x.T.reshape(-1).astype(jnp.int32)
    w1t = W1.T
    b1r = b1.reshape(1, D)
    b2c = b2.reshape(N_TAGS, 1)
    gs = [_sc_gather(lax.dynamic_slice(idx, (k * CN,), (CN,)), emb)
          for k in range(NCHUNK)]
    out_t = None
    for k in range(NCHUNK):
        out_t = _tc_mlp_chunk(k, gs[k], w1t, b1r, W2, b2c, out_t)
    return lax.transpose(out_t, (2, 1, 0))


# speedup vs baseline: 1.0793x; 1.0182x over previous
import functools

import jax
import jax.numpy as jnp
from jax import lax
from jax.experimental import pallas as pl
from jax.experimental.pallas import tpu as pltpu
from jax.experimental.pallas import tpu_sc as plsc

B, L = 1024, 200
N = B * L
D = 128
N_TAGS = 45

NCHUNK = 5
CL = L // NCHUNK
CN = CL * B

_info = plsc.get_sparse_core_info()
NC, NS = _info.num_cores, _info.num_subcores
NW = NC * NS
PER_W = CN // NW
CH = 128
N_CH = PER_W // CH


NBUF = 4
DP = 3


def _sc_gather_kernel(idx_hbm, table_hbm, out_hbm, idx_v, *scr):
    bufs, gsems, osems = scr[:NBUF], scr[NBUF:2 * NBUF], scr[2 * NBUF:]
    wid = lax.axis_index("s") * NC + lax.axis_index("c")
    base = wid * PER_W
    pltpu.sync_copy(idx_hbm.at[pl.ds(base, PER_W)], idx_v)

    def start_gather(j):
        b = j % NBUF
        pltpu.async_copy(
            table_hbm.at[idx_v.at[pl.ds(j * CH, CH)]], bufs[b], gsems[b]
        )

    for j in range(min(DP, N_CH)):
        start_gather(j)

    for j in range(N_CH):
        b = j % NBUF
        pltpu.make_async_copy(
            table_hbm.at[idx_v.at[pl.ds(j * CH, CH)]], bufs[b], gsems[b]
        ).wait()
        pltpu.async_copy(bufs[b], out_hbm.at[pl.ds(base + j * CH, CH)], osems[b])
        nj = j + DP
        if DP <= nj < N_CH:
            bn = nj % NBUF
            if nj >= NBUF:
                pltpu.make_async_copy(
                    bufs[bn],
                    out_hbm.at[pl.ds(base + (nj - NBUF) * CH, CH)],
                    osems[bn],
                ).wait()
            start_gather(nj)

    for j in range(max(0, N_CH - NBUF), N_CH):
        b = j % NBUF
        pltpu.make_async_copy(
            bufs[b], out_hbm.at[pl.ds(base + j * CH, CH)], osems[b]
        ).wait()


def _sc_gather(idx, table):
    mesh = plsc.VectorSubcoreMesh(core_axis_name="c", subcore_axis_name="s")
    k = functools.partial(
        pl.kernel,
        mesh=mesh,
        out_type=jax.ShapeDtypeStruct((CN, D), jnp.float32),
        scratch_types=[
            pltpu.VMEM((PER_W,), jnp.int32),
        ]
        + [pltpu.VMEM((CH, D), jnp.float32) for _ in range(NBUF)]
        + [pltpu.SemaphoreType.DMA for _ in range(2 * NBUF)],
    )(_sc_gather_kernel)
    return k(idx, table)


BL = 8


def _mlp_body(g_ref, w1t_ref, b1_ref, w2_ref, b2_ref, out_ref):
    h = jnp.maximum(
        jnp.dot(
            g_ref[...], w1t_ref[...], preferred_element_type=jnp.float32
        )
        + b1_ref[...],
        0.0,
    )
    for j in range(BL):
        ot = (
            lax.dot_general(
                w2_ref[...], h[j * B:(j + 1) * B, :], (((1,), (1,)), ((), ())),
                preferred_element_type=jnp.float32,
            )
            + b2_ref[...]
        )
        out_ref[:, j, :] = ot


def _mlp_kernel_first(g_ref, w1t_ref, b1_ref, w2_ref, b2_ref, out_ref):
    _mlp_body(g_ref, w1t_ref, b1_ref, w2_ref, b2_ref, out_ref)


def _mlp_kernel_acc(g_ref, w1t_ref, b1_ref, w2_ref, b2_ref, acc_ref, out_ref):
    _mlp_body(g_ref, w1t_ref, b1_ref, w2_ref, b2_ref, out_ref)


def _tc_mlp_chunk(k, g, w1t, b1, w2, b2c, acc):
    k0 = k * (CL // BL)
    in_specs = [
        pl.BlockSpec((BL * B, D), lambda i: (i, 0)),
        pl.BlockSpec((D, D), lambda i: (0, 0)),
        pl.BlockSpec((1, D), lambda i: (0, 0)),
        pl.BlockSpec((N_TAGS, D), lambda i: (0, 0)),
        pl.BlockSpec((N_TAGS, 1), lambda i: (0, 0)),
    ]
    args = (g, w1t, b1, w2, b2c)
    body = _mlp_kernel_first
    kwargs = {}
    if acc is not None:
        in_specs.append(pl.BlockSpec(memory_space=pl.ANY))
        args = args + (acc,)
        body = _mlp_kernel_acc
        kwargs = dict(input_output_aliases={5: 0})
    return pl.pallas_call(
        body,
        grid=(CL // BL,),
        in_specs=in_specs,
        out_specs=pl.BlockSpec((N_TAGS, BL, B), lambda i: (0, k0 + i, 0)),
        out_shape=jax.ShapeDtypeStruct((N_TAGS, L, B), jnp.float32),
        **kwargs,
    )(*args)


def kernel(x, emb, W1, b1, W2, b2):
    idx = x.T.reshape(-1).astype(jnp.int32)
    w1t = W1.T
    b1r = b1.reshape(1, D)
    b2c = b2.reshape(N_TAGS, 1)
    gs = [_sc_gather(lax.dynamic_slice(idx, (k * CN,), (CN,)), emb)
          for k in range(NCHUNK)]
    out_t = None
    for k in range(NCHUNK):
        out_t = _tc_mlp_chunk(k, gs[k], w1t, b1r, W2, b2c, out_t)
    return lax.transpose(out_t, (2, 1, 0))


# speedup vs baseline: 1.0973x; 1.0167x over previous
import functools

import jax
import jax.numpy as jnp
from jax import lax
from jax.experimental import pallas as pl
from jax.experimental.pallas import tpu as pltpu
from jax.experimental.pallas import tpu_sc as plsc

B, L = 1024, 200
N = B * L
D = 128
N_TAGS = 45

NCHUNK = 5
CL = L // NCHUNK
CN = CL * B

_info = plsc.get_sparse_core_info()
NC, NS = _info.num_cores, _info.num_subcores
NW = NC * NS
PER_W = CN // NW
CH = 128
N_CH = PER_W // CH


NBUF = 6
DP = 4


def _sc_gather_kernel(idx_hbm, table_hbm, out_hbm, idx_v, *scr):
    bufs, gsems, osems = scr[:NBUF], scr[NBUF:2 * NBUF], scr[2 * NBUF:]
    wid = lax.axis_index("s") * NC + lax.axis_index("c")
    base = wid * PER_W
    pltpu.sync_copy(idx_hbm.at[pl.ds(base, PER_W)], idx_v)

    def start_gather(j):
        b = j % NBUF
        pltpu.async_copy(
            table_hbm.at[idx_v.at[pl.ds(j * CH, CH)]], bufs[b], gsems[b]
        )

    for j in range(min(DP, N_CH)):
        start_gather(j)

    for j in range(N_CH):
        b = j % NBUF
        pltpu.make_async_copy(
            table_hbm.at[idx_v.at[pl.ds(j * CH, CH)]], bufs[b], gsems[b]
        ).wait()
        pltpu.async_copy(bufs[b], out_hbm.at[pl.ds(base + j * CH, CH)], osems[b])
        nj = j + DP
        if DP <= nj < N_CH:
            bn = nj % NBUF
            if nj >= NBUF:
                pltpu.make_async_copy(
                    bufs[bn],
                    out_hbm.at[pl.ds(base + (nj - NBUF) * CH, CH)],
                    osems[bn],
                ).wait()
            start_gather(nj)

    for j in range(max(0, N_CH - NBUF), N_CH):
        b = j % NBUF
        pltpu.make_async_copy(
            bufs[b], out_hbm.at[pl.ds(base + j * CH, CH)], osems[b]
        ).wait()


def _sc_gather(idx, table):
    mesh = plsc.VectorSubcoreMesh(core_axis_name="c", subcore_axis_name="s")
    k = functools.partial(
        pl.kernel,
        mesh=mesh,
        out_type=jax.ShapeDtypeStruct((CN, D), jnp.float32),
        scratch_types=[
            pltpu.VMEM((PER_W,), jnp.int32),
        ]
        + [pltpu.VMEM((CH, D), jnp.float32) for _ in range(NBUF)]
        + [pltpu.SemaphoreType.DMA for _ in range(2 * NBUF)],
    )(_sc_gather_kernel)
    return k(idx, table)


BL = 8


def _mlp_body(g_ref, w1t_ref, b1_ref, w2_ref, b2_ref, out_ref):
    h = jnp.maximum(
        jnp.dot(
            g_ref[...], w1t_ref[...], preferred_element_type=jnp.float32
        )
        + b1_ref[...],
        0.0,
    )
    for j in range(BL):
        ot = (
            lax.dot_general(
                w2_ref[...], h[j * B:(j + 1) * B, :], (((1,), (1,)), ((), ())),
                preferred_element_type=jnp.float32,
            )
            + b2_ref[...]
        )
        out_ref[:, j, :] = ot


def _mlp_kernel_first(g_ref, w1t_ref, b1_ref, w2_ref, b2_ref, out_ref):
    _mlp_body(g_ref, w1t_ref, b1_ref, w2_ref, b2_ref, out_ref)


def _mlp_kernel_acc(g_ref, w1t_ref, b1_ref, w2_ref, b2_ref, acc_ref, out_ref):
    _mlp_body(g_ref, w1t_ref, b1_ref, w2_ref, b2_ref, out_ref)


def _tc_mlp_chunk(k, g, w1t, b1, w2, b2c, acc):
    k0 = k * (CL // BL)
    in_specs = [
        pl.BlockSpec((BL * B, D), lambda i: (i, 0)),
        pl.BlockSpec((D, D), lambda i: (0, 0)),
        pl.BlockSpec((1, D), lambda i: (0, 0)),
        pl.BlockSpec((N_TAGS, D), lambda i: (0, 0)),
        pl.BlockSpec((N_TAGS, 1), lambda i: (0, 0)),
    ]
    args = (g, w1t, b1, w2, b2c)
    body = _mlp_kernel_first
    kwargs = {}
    if acc is not None:
        in_specs.append(pl.BlockSpec(memory_space=pl.ANY))
        args = args + (acc,)
        body = _mlp_kernel_acc
        kwargs = dict(input_output_aliases={5: 0})
    return pl.pallas_call(
        body,
        grid=(CL // BL,),
        in_specs=in_specs,
        out_specs=pl.BlockSpec((N_TAGS, BL, B), lambda i: (0, k0 + i, 0)),
        out_shape=jax.ShapeDtypeStruct((N_TAGS, L, B), jnp.float32),
        **kwargs,
    )(*args)


def kernel(x, emb, W1, b1, W2, b2):
    idx = x.T.reshape(-1).astype(jnp.int32)
    w1t = W1.T
    b1r = b1.reshape(1, D)
    b2c = b2.reshape(N_TAGS, 1)
    gs = [_sc_gather(lax.dynamic_slice(idx, (k * CN,), (CN,)), emb)
          for k in range(NCHUNK)]
    out_t = None
    for k in range(NCHUNK):
        out_t = _tc_mlp_chunk(k, gs[k], w1t, b1r, W2, b2c, out_t)
    return lax.transpose(out_t, (2, 1, 0))
